# Initial kernel scaffold; baseline (speedup 1.0000x reference)
import jax
import jax.numpy as jnp
from jax.experimental import pallas as pl


def kernel(x, adj, W, b):
    raise NotImplementedError("write your pallas kernel here")


# speedup vs baseline: 1.6106x; 1.6106x over previous
import functools

import jax
import jax.numpy as jnp
from jax.experimental import pallas as pl

MIN_NORM = 1e-15
EPS = 4e-3
C = 1.0


def _row_norm(v):
    return jnp.maximum(jnp.sqrt(jnp.sum(v * v, axis=-1, keepdims=True)), MIN_NORM)


def _artanh(z):
    z = jnp.clip(z, -1.0 + 1e-7, 1.0 - 1e-7)
    return 0.5 * (jnp.log1p(z) - jnp.log1p(-z))


def _proj(v):
    norm = _row_norm(v)
    maxnorm = 1.0 - EPS
    return jnp.where(norm > maxnorm, v / norm * maxnorm, v)


def _expmap0(u):
    u_norm = _row_norm(u)
    return jnp.tanh(u_norm) * u / u_norm


def _logmap0(p):
    p_norm = _row_norm(p)
    return _artanh(p_norm) * p / p_norm


def _stage1_body(x_ref, w_ref, b_ref, xt_ref):
    x = x_ref[...]
    w = w_ref[...]
    b = b_ref[...]

    x_hyp = _proj(_expmap0(x))

    x_norm = _row_norm(x_hyp)
    mx = jnp.dot(x_hyp, w.T, preferred_element_type=jnp.float32)
    mx_norm = _row_norm(mx)
    res_c = jnp.tanh(mx_norm / x_norm * _artanh(x_norm)) * mx / mx_norm
    cond = jnp.all(mx == 0.0, axis=-1, keepdims=True)
    mv = _proj(jnp.where(cond, jnp.zeros_like(res_c), res_c))

    hyp_bias = _proj(_expmap0(b))
    x2 = jnp.sum(mv * mv, axis=-1, keepdims=True)
    y2 = jnp.sum(hyp_bias * hyp_bias, axis=-1, keepdims=True)
    xy = jnp.sum(mv * hyp_bias, axis=-1, keepdims=True)
    num = (1.0 + 2.0 * xy + y2) * mv + (1.0 - x2) * hyp_bias
    denom = 1.0 + 2.0 * xy + x2 * y2
    h = _proj(num / jnp.maximum(denom, MIN_NORM))

    xt_ref[...] = _logmap0(h)


def _stage2_body(adj_ref, xt_ref, out_ref):
    support = jnp.dot(adj_ref[...], xt_ref[...], preferred_element_type=jnp.float32)
    h = _proj(_expmap0(support))
    t = jax.nn.relu(_logmap0(h))
    out_ref[...] = _proj(_expmap0(t))


def _pick_block(n, target):
    best = n
    for r in range(8, min(n, target) + 1, 8):
        if n % r == 0:
            best = r
    return best if n % best == 0 else n


@jax.jit
def kernel(x, adj, W, b):
    n, d = x.shape

    r1 = _pick_block(n, 2000)
    xt = pl.pallas_call(
        _stage1_body,
        grid=(n // r1,),
        in_specs=[
            pl.BlockSpec((r1, d), lambda i: (i, 0)),
            pl.BlockSpec((d, d), lambda i: (0, 0)),
            pl.BlockSpec((1, d), lambda i: (0, 0)),
        ],
        out_specs=pl.BlockSpec((r1, d), lambda i: (i, 0)),
        out_shape=jax.ShapeDtypeStruct((n, d), jnp.float32),
    )(x, W, b.reshape(1, d))

    r2 = _pick_block(n, 400)
    out = pl.pallas_call(
        _stage2_body,
        grid=(n // r2,),
        in_specs=[
            pl.BlockSpec((r2, n), lambda i: (i, 0)),
            pl.BlockSpec((n, d), lambda i: (0, 0)),
        ],
        out_specs=pl.BlockSpec((r2, d), lambda i: (i, 0)),
        out_shape=jax.ShapeDtypeStruct((n, d), jnp.float32),
    )(adj, xt)
    return out


# speedup vs baseline: 1.6167x; 1.0038x over previous
import functools

import jax
import jax.numpy as jnp
from jax.experimental import pallas as pl

MIN_NORM = 1e-15
EPS = 4e-3
C = 1.0


def _row_norm(v):
    return jnp.maximum(jnp.sqrt(jnp.sum(v * v, axis=-1, keepdims=True)), MIN_NORM)


def _artanh(z):
    z = jnp.clip(z, -1.0 + 1e-7, 1.0 - 1e-7)
    return 0.5 * (jnp.log1p(z) - jnp.log1p(-z))


def _proj(v):
    norm = _row_norm(v)
    maxnorm = 1.0 - EPS
    return jnp.where(norm > maxnorm, v / norm * maxnorm, v)


def _expmap0(u):
    u_norm = _row_norm(u)
    return jnp.tanh(u_norm) * u / u_norm


def _logmap0(p):
    p_norm = _row_norm(p)
    return _artanh(p_norm) * p / p_norm


def _stage1_body(x_ref, w_ref, b_ref, xt_ref):
    x = x_ref[...]
    w = w_ref[...]
    b = b_ref[...]

    x_hyp = _proj(_expmap0(x))

    x_norm = _row_norm(x_hyp)
    mx = jnp.dot(x_hyp, w.T, preferred_element_type=jnp.float32)
    mx_norm = _row_norm(mx)
    res_c = jnp.tanh(mx_norm / x_norm * _artanh(x_norm)) * mx / mx_norm
    cond = jnp.all(mx == 0.0, axis=-1, keepdims=True)
    mv = _proj(jnp.where(cond, jnp.zeros_like(res_c), res_c))

    hyp_bias = _proj(_expmap0(b))
    x2 = jnp.sum(mv * mv, axis=-1, keepdims=True)
    y2 = jnp.sum(hyp_bias * hyp_bias, axis=-1, keepdims=True)
    xy = jnp.sum(mv * hyp_bias, axis=-1, keepdims=True)
    num = (1.0 + 2.0 * xy + y2) * mv + (1.0 - x2) * hyp_bias
    denom = 1.0 + 2.0 * xy + x2 * y2
    h = _proj(num / jnp.maximum(denom, MIN_NORM))

    xt_ref[...] = _logmap0(h)


def _stage2_body(adj_ref, xt_ref, out_ref):
    a = adj_ref[...].astype(jnp.bfloat16)
    v = xt_ref[...].astype(jnp.bfloat16)
    support = jnp.dot(a, v, preferred_element_type=jnp.float32)
    h = _proj(_expmap0(support))
    t = jax.nn.relu(_logmap0(h))
    out_ref[...] = _proj(_expmap0(t))


def _pick_block(n, target):
    best = n
    for r in range(8, min(n, target) + 1, 8):
        if n % r == 0:
            best = r
    return best if n % best == 0 else n


@jax.jit
def kernel(x, adj, W, b):
    n, d = x.shape

    r1 = _pick_block(n, 2000)
    xt = pl.pallas_call(
        _stage1_body,
        grid=(n // r1,),
        in_specs=[
            pl.BlockSpec((r1, d), lambda i: (i, 0)),
            pl.BlockSpec((d, d), lambda i: (0, 0)),
            pl.BlockSpec((1, d), lambda i: (0, 0)),
        ],
        out_specs=pl.BlockSpec((r1, d), lambda i: (i, 0)),
        out_shape=jax.ShapeDtypeStruct((n, d), jnp.float32),
    )(x, W, b.reshape(1, d))

    r2 = _pick_block(n, 400)
    out = pl.pallas_call(
        _stage2_body,
        grid=(n // r2,),
        in_specs=[
            pl.BlockSpec((r2, n), lambda i: (i, 0)),
            pl.BlockSpec((n, d), lambda i: (0, 0)),
        ],
        out_specs=pl.BlockSpec((r2, d), lambda i: (i, 0)),
        out_shape=jax.ShapeDtypeStruct((n, d), jnp.float32),
    )(adj, xt)
    return out


# speedup vs baseline: 1.6318x; 1.0093x over previous
import jax
import jax.numpy as jnp
from jax.experimental import pallas as pl
from jax.experimental.pallas import tpu as pltpu

MIN_NORM = 1e-15
EPS = 4e-3
C = 1.0


def _row_norm(v):
    return jnp.maximum(jnp.sqrt(jnp.sum(v * v, axis=-1, keepdims=True)), MIN_NORM)


def _artanh(z):
    z = jnp.clip(z, -1.0 + 1e-7, 1.0 - 1e-7)
    return 0.5 * (jnp.log1p(z) - jnp.log1p(-z))


def _proj(v):
    norm = _row_norm(v)
    maxnorm = 1.0 - EPS
    return jnp.where(norm > maxnorm, v / norm * maxnorm, v)


def _expmap0(u):
    u_norm = _row_norm(u)
    return jnp.tanh(u_norm) * u / u_norm


def _logmap0(p):
    p_norm = _row_norm(p)
    return _artanh(p_norm) * p / p_norm


def _tangent_features(x, w, b):
    x_hyp = _proj(_expmap0(x))

    x_norm = _row_norm(x_hyp)
    mx = jnp.dot(x_hyp, w.T, preferred_element_type=jnp.float32)
    mx_norm = _row_norm(mx)
    res_c = jnp.tanh(mx_norm / x_norm * _artanh(x_norm)) * mx / mx_norm
    cond = jnp.all(mx == 0.0, axis=-1, keepdims=True)
    mv = _proj(jnp.where(cond, jnp.zeros_like(res_c), res_c))

    hyp_bias = _proj(_expmap0(b))
    x2 = jnp.sum(mv * mv, axis=-1, keepdims=True)
    y2 = jnp.sum(hyp_bias * hyp_bias, axis=-1, keepdims=True)
    xy = jnp.sum(mv * hyp_bias, axis=-1, keepdims=True)
    num = (1.0 + 2.0 * xy + y2) * mv + (1.0 - x2) * hyp_bias
    denom = 1.0 + 2.0 * xy + x2 * y2
    h = _proj(num / jnp.maximum(denom, MIN_NORM))

    return _logmap0(h)


def _body(x_ref, w_ref, b_ref, adj_ref, out_ref, xt_ref):
    @pl.when(pl.program_id(0) == 0)
    def _():
        xt = _tangent_features(x_ref[...], w_ref[...], b_ref[...])
        xt_ref[...] = xt.astype(jnp.bfloat16)

    a = adj_ref[...].astype(jnp.bfloat16)
    support = jnp.dot(a, xt_ref[...], preferred_element_type=jnp.float32)
    h = _proj(_expmap0(support))
    t = jax.nn.relu(_logmap0(h))
    out_ref[...] = _proj(_expmap0(t))


def _pick_block(n, target):
    best = n
    for r in range(8, min(n, target) + 1, 8):
        if n % r == 0:
            best = r
    return best if n % best == 0 else n


@jax.jit
def kernel(x, adj, W, b):
    n, d = x.shape
    r = _pick_block(n, 400)
    return pl.pallas_call(
        _body,
        grid=(n // r,),
        in_specs=[
            pl.BlockSpec((n, d), lambda i: (0, 0)),
            pl.BlockSpec((d, d), lambda i: (0, 0)),
            pl.BlockSpec((1, d), lambda i: (0, 0)),
            pl.BlockSpec((r, n), lambda i: (i, 0)),
        ],
        out_specs=pl.BlockSpec((r, d), lambda i: (i, 0)),
        out_shape=jax.ShapeDtypeStruct((n, d), jnp.float32),
        scratch_shapes=[pltpu.VMEM((n, d), jnp.bfloat16)],
    )(x, W, b.reshape(1, d), adj)
